# Initial kernel scaffold; baseline (speedup 1.0000x reference)
import jax
import jax.numpy as jnp
from jax.experimental import pallas as pl


def kernel(x, edge_index, Wq0, bq0, Wk0, bk0, Wv0, bv0, Ws0, bs0, Wq1, bq1, Wk1, bk1, Wv1, bv1, Ws1, bs1, Wq2, bq2, Wk2, bk2, Wv2, bv2, Ws2, bs2, Wq3, bq3, Wk3, bk3, Wv3, bv3, Ws3, bs3):
    raise NotImplementedError("write your pallas kernel here")


# speedup vs baseline: 7.2684x; 7.2684x over previous
import functools
import math

import jax
import jax.numpy as jnp
from jax import lax
from jax.experimental import pallas as pl
from jax.experimental.pallas import tpu as pltpu
from jax.experimental.pallas import tpu_sc as plsc

NN = 10000
EE = 320000
H = 8
NSUB = 16
NCORE = 2
B = 80
EC = EE // NSUB
NB = EC // B
G = B // 16
RPS = 624
CHUNKS = [(i * B, B) for i in range(RPS // B)] + [(RPS - RPS % B, RPS % B)]
W = 128
HPC = 4
DIMS_L = [(128, 256), (256, 256), (256, 256), (256, 128)]


def _make_proj(din, hc):
    D = hc // 2
    bn = 400
    grid = NN // bn
    wdim = 4 * hc

    def body(x_ref, w_ref, b_ref, q_ref, k_ref, v_ref, s_ref):
        acc = jnp.dot(x_ref[...], w_ref[...],
                      preferred_element_type=jnp.float32) + b_ref[...]
        pad = jnp.zeros((bn, W - D), jnp.float32) if D < W else None
        for t, ref in enumerate([q_ref, k_ref, v_ref]):
            blk = acc[:, t * hc:(t + 1) * hc]
            lo, hi = blk[:, :D], blk[:, D:]
            if pad is not None:
                lo = jnp.concatenate([lo, pad], axis=1)
                hi = jnp.concatenate([hi, pad], axis=1)
            ref[0] = lo
            ref[1] = hi
        s_ref[...] = acc[:, 3 * hc:]

    return pl.pallas_call(
        body,
        grid=(grid,),
        in_specs=[
            pl.BlockSpec((bn, din), lambda i: (i, 0)),
            pl.BlockSpec((din, wdim), lambda i: (0, 0)),
            pl.BlockSpec((1, wdim), lambda i: (0, 0)),
        ],
        out_specs=[
            pl.BlockSpec((2, bn, W), lambda i: (0, i, 0)),
            pl.BlockSpec((2, bn, W), lambda i: (0, i, 0)),
            pl.BlockSpec((2, bn, W), lambda i: (0, i, 0)),
            pl.BlockSpec((bn, hc), lambda i: (i, 0)),
        ],
        out_shape=[
            jax.ShapeDtypeStruct((2, NN, W), jnp.float32),
            jax.ShapeDtypeStruct((2, NN, W), jnp.float32),
            jax.ShapeDtypeStruct((2, NN, W), jnp.float32),
            jax.ShapeDtypeStruct((NN, hc), jnp.float32),
        ],
    )


def _make_edge(hc):
    CH = hc // 2
    c = hc // H
    inv = 1.0 / math.sqrt(c)
    mesh = plsc.VectorSubcoreMesh(core_axis_name="c", subcore_axis_name="s")

    @functools.partial(
        pl.kernel,
        mesh=mesh,
        compiler_params=pltpu.CompilerParams(needs_layout_passes=False),
        out_type=(
            jax.ShapeDtypeStruct((NCORE * NN, W), jnp.float32),
            jax.ShapeDtypeStruct((NCORE * NN,), jnp.float32),
            jax.ShapeDtypeStruct((NCORE * NN,), jnp.float32),
            jax.ShapeDtypeStruct((NCORE * NN,), jnp.float32),
            jax.ShapeDtypeStruct((NCORE * NN,), jnp.float32),
        ),
        scratch_types=[
            pltpu.VMEM((B, W), jnp.float32),
            pltpu.VMEM((B, W), jnp.float32),
            pltpu.VMEM((B, W), jnp.float32),
            pltpu.VMEM((B, W), jnp.float32),
            pltpu.VMEM((HPC, B), jnp.float32),
            pltpu.VMEM((B,), jnp.int32),
            pltpu.VMEM((B,), jnp.int32),
            pltpu.VMEM((B,), jnp.int32),
            pltpu.VMEM_SHARED((NN, W), jnp.float32),
            pltpu.VMEM_SHARED((NN,), jnp.float32),
            pltpu.VMEM_SHARED((NN,), jnp.float32),
            pltpu.VMEM_SHARED((NN,), jnp.float32),
            pltpu.VMEM_SHARED((NN,), jnp.float32),
            pltpu.SemaphoreType.DMA,
        ],
    )
    def edge_kernel(q_hbm, k_hbm, v_hbm, src_hbm, dst_hbm,
                    num_out, den_out0, den_out1, den_out2, den_out3,
                    qb, kb, vb, msg, denT, dstv, dstg, srcg,
                    num_sp, den_sp0, den_sp1, den_sp2, den_sp3, sem):
        den_sps = [den_sp0, den_sp1, den_sp2, den_sp3]
        den_outs = [den_out0, den_out1, den_out2, den_out3]
        core = lax.axis_index("c")
        sub = lax.axis_index("s")
        lane = lax.iota(jnp.int32, 16)
        zero16 = jnp.zeros((16,), jnp.float32)

        def zrow(r, carry):
            for j in range(W // 16):
                msg[r, pl.ds(j * 16, 16)] = zero16
            return carry
        lax.fori_loop(0, B, zrow, 0)
        for h in range(HPC):
            for j in range(B // 16):
                denT[h, pl.ds(j * 16, 16)] = zero16

        rbase = sub * RPS
        for start, rows in CHUNKS:
            pltpu.sync_copy(msg.at[pl.ds(0, rows)],
                            num_sp.at[pl.ds(rbase + start, rows)])
            for h in range(HPC):
                pltpu.sync_copy(denT.at[h].at[pl.ds(0, rows)],
                                den_sps[h].at[pl.ds(rbase + start, rows)])

        @pl.when(sub == NSUB - 1)
        def _init_tail():
            pltpu.sync_copy(msg.at[pl.ds(0, 16)],
                            num_sp.at[pl.ds(NSUB * RPS, 16)])
            for h in range(HPC):
                pltpu.sync_copy(denT.at[h].at[pl.ds(0, 16)],
                                den_sps[h].at[pl.ds(NSUB * RPS, 16)])

        plsc.subcore_barrier()

        coff = core * NN
        ebase = sub * EC

        def block_body(b, carry):
            off = ebase + b * B
            pltpu.sync_copy(src_hbm.at[pl.ds(off, B)], srcg)
            pltpu.sync_copy(dst_hbm.at[pl.ds(off, B)], dstv)
            for i in range(B // 16):
                sl = pl.ds(i * 16, 16)
                srcg[sl] = srcg[sl] + coff
                dstg[sl] = dstv[sl] + coff
            cq = pltpu.async_copy(q_hbm.at[dstg], qb, sem)
            ck = pltpu.async_copy(k_hbm.at[srcg], kb, sem)
            cv = pltpu.async_copy(v_hbm.at[srcg], vb, sem)
            cq.wait()
            ck.wait()
            cv.wait()

            def group_body(g, gcarry):
                row = g * 16 + lane
                exs = []
                for h in range(HPC):
                    acc = zero16
                    for cc in range(c):
                        colv = jnp.full((16,), h * c + cc, jnp.int32)
                        qv = plsc.load_gather(qb, [row, colv])
                        kv = plsc.load_gather(kb, [row, colv])
                        acc = acc + qv * kv
                    ex = jnp.exp(acc * inv)
                    exs.append(ex)
                    denT[h, pl.ds(g * 16, 16)] = ex
                for ch in range(CH):
                    colv = jnp.full((16,), ch, jnp.int32)
                    mv = plsc.load_gather(vb, [row, colv])
                    plsc.store_scatter(msg, [row, colv], mv * exs[ch // c])
                return gcarry
            lax.fori_loop(0, G, group_body, 0)

            pltpu.sync_copy(msg, num_sp.at[dstv], add=True)
            for h in range(HPC):
                pltpu.sync_copy(denT.at[h], den_sps[h].at[dstv], add=True)
            return carry
        lax.fori_loop(0, NB, block_body, 0)

        plsc.subcore_barrier()

        obase = coff + rbase
        for start, rows in CHUNKS:
            pltpu.sync_copy(num_sp.at[pl.ds(rbase + start, rows)],
                            msg.at[pl.ds(0, rows)])
            pltpu.sync_copy(msg.at[pl.ds(0, rows)],
                            num_out.at[pl.ds(obase + start, rows)])
            for h in range(HPC):
                pltpu.sync_copy(den_sps[h].at[pl.ds(rbase + start, rows)],
                                denT.at[h].at[pl.ds(0, rows)])
                pltpu.sync_copy(denT.at[h].at[pl.ds(0, rows)],
                                den_outs[h].at[pl.ds(obase + start, rows)])

        @pl.when(sub == NSUB - 1)
        def _out_tail():
            pltpu.sync_copy(num_sp.at[pl.ds(NSUB * RPS, 16)],
                            msg.at[pl.ds(0, 16)])
            pltpu.sync_copy(msg.at[pl.ds(0, 16)],
                            num_out.at[pl.ds(coff + NSUB * RPS, 16)])
            for h in range(HPC):
                pltpu.sync_copy(den_sps[h].at[pl.ds(NSUB * RPS, 16)],
                                denT.at[h].at[pl.ds(0, 16)])
                pltpu.sync_copy(denT.at[h].at[pl.ds(0, 16)],
                                den_outs[h].at[pl.ds(coff + NSUB * RPS, 16)])

    return edge_kernel


def _make_final(hc, use_gelu):
    D = hc // 2
    c = hc // H
    bn = 400
    grid = NN // bn

    def body(num_ref, den_ref, skip_ref, out_ref):
        ih = lax.broadcasted_iota(jnp.int32, (HPC, D), 0)
        ic = lax.broadcasted_iota(jnp.int32, (HPC, D), 1)
        R = (ic // c == ih).astype(jnp.float32)
        halves = []
        for half in range(2):
            dexp = jnp.dot(den_ref[half], R,
                           preferred_element_type=jnp.float32)
            halves.append(num_ref[half, :, :D] / (dexp + 1e-16))
        out = jnp.concatenate(halves, axis=1) + skip_ref[...]
        if use_gelu:
            out = 0.5 * out * (1.0 + lax.erf(out * (1.0 / math.sqrt(2.0))))
        out_ref[...] = out

    return pl.pallas_call(
        body,
        grid=(grid,),
        in_specs=[
            pl.BlockSpec((2, bn, W), lambda i: (0, i, 0)),
            pl.BlockSpec((2, bn, HPC), lambda i: (0, i, 0)),
            pl.BlockSpec((bn, hc), lambda i: (i, 0)),
        ],
        out_specs=pl.BlockSpec((bn, hc), lambda i: (i, 0)),
        out_shape=jax.ShapeDtypeStruct((NN, hc), jnp.float32),
    )


_PROJ = {}
_EDGE = {}
_FINAL = {}
for _l, (_din, _hc) in enumerate(DIMS_L):
    if (_din, _hc) not in _PROJ:
        _PROJ[(_din, _hc)] = _make_proj(_din, _hc)
    if _hc not in _EDGE:
        _EDGE[_hc] = _make_edge(_hc)
    if (_hc, _l < 3) not in _FINAL:
        _FINAL[(_hc, _l < 3)] = _make_final(_hc, _l < 3)


def kernel(x, edge_index,
           Wq0, bq0, Wk0, bk0, Wv0, bv0, Ws0, bs0,
           Wq1, bq1, Wk1, bk1, Wv1, bv1, Ws1, bs1,
           Wq2, bq2, Wk2, bk2, Wv2, bv2, Ws2, bs2,
           Wq3, bq3, Wk3, bk3, Wv3, bv3, Ws3, bs3):
    params = (Wq0, bq0, Wk0, bk0, Wv0, bv0, Ws0, bs0,
              Wq1, bq1, Wk1, bk1, Wv1, bv1, Ws1, bs1,
              Wq2, bq2, Wk2, bk2, Wv2, bv2, Ws2, bs2,
              Wq3, bq3, Wk3, bk3, Wv3, bv3, Ws3, bs3)
    srcp = edge_index[0]
    dstp = edge_index[1]
    h = x
    for l, (din, hc) in enumerate(DIMS_L):
        Wq, bq, Wk, bk, Wv, bv, Ws, bs = params[8 * l:8 * (l + 1)]
        Wc = jnp.concatenate([Wq, Wk, Wv, Ws], axis=1)
        bc = jnp.concatenate([bq, bk, bv, bs]).reshape(1, -1)
        q3, k3, v3, skip = _PROJ[(din, hc)](h, Wc, bc)
        num, d0, d1, d2, d3 = _EDGE[hc](
            q3.reshape(2 * NN, W), k3.reshape(2 * NN, W),
            v3.reshape(2 * NN, W), srcp, dstp)
        den = jnp.stack([d0, d1, d2, d3], axis=-1).reshape(2, NN, HPC)
        h = _FINAL[(hc, l < 3)](num.reshape(2, NN, W), den, skip)
    return h
